# Initial kernel scaffold; baseline (speedup 1.0000x reference)
import jax
import jax.numpy as jnp
from jax.experimental import pallas as pl


def kernel(x, edge_index, batch, emb, W1, b1, W2, b2, cW1, cb1, cW2, cb2):
    raise NotImplementedError("write your pallas kernel here")


# speedup vs baseline: 3.0252x; 3.0252x over previous
import functools

import jax
import jax.numpy as jnp
from jax.experimental import pallas as pl
from jax.experimental.pallas import tpu as pltpu

N = 50000
G = 128
H = 128


def _pool_mlp_body(batch_ref, h_ref, cW1_ref, cb1_ref, cW2_ref, cb2_ref,
                   out_ref, acc_ref, cnt_ref):
    i = pl.program_id(0)
    nsteps = pl.num_programs(0)

    @pl.when(i == 0)
    def _init():
        acc_ref[...] = jnp.zeros_like(acc_ref)
        cnt_ref[...] = jnp.zeros_like(cnt_ref)

    b = batch_ref[0, 0, :]
    seg = jax.lax.broadcasted_iota(jnp.int32, (G, 1), 0)
    onehot = jnp.where(b[None, :] == seg, 1.0, 0.0)
    h = h_ref[...]
    acc_ref[...] += jnp.dot(onehot, h, preferred_element_type=jnp.float32)
    cnt_ref[...] += jnp.sum(onehot, axis=1, keepdims=True)

    @pl.when(i == nsteps - 1)
    def _fin():
        g = acc_ref[...] / jnp.maximum(cnt_ref[...], 1.0)
        z = jnp.maximum(
            jnp.dot(g, cW1_ref[...], preferred_element_type=jnp.float32)
            + cb1_ref[...][None, :], 0.0)
        out_ref[...] = (jnp.dot(z, cW2_ref[...],
                                preferred_element_type=jnp.float32)
                        + cb2_ref[...][None, :])


def _pool_mlp(batch, h, cW1, cb1, cW2, cb2):
    CHUNK = 2000
    nsteps = N // CHUNK
    batch2 = batch.reshape(nsteps, 1, CHUNK).astype(jnp.int32)
    return pl.pallas_call(
        _pool_mlp_body,
        grid=(nsteps,),
        in_specs=[
            pl.BlockSpec((1, 1, CHUNK), lambda i: (i, 0, 0)),
            pl.BlockSpec((CHUNK, H), lambda i: (i, 0)),
            pl.BlockSpec((H, H // 2), lambda i: (0, 0)),
            pl.BlockSpec((H // 2,), lambda i: (0,)),
            pl.BlockSpec((H // 2, 1), lambda i: (0, 0)),
            pl.BlockSpec((1,), lambda i: (0,)),
        ],
        out_specs=pl.BlockSpec((G, 1), lambda i: (0, 0)),
        out_shape=jax.ShapeDtypeStruct((G, 1), jnp.float32),
        scratch_shapes=[
            pltpu.VMEM((G, H), jnp.float32),
            pltpu.VMEM((G, 1), jnp.float32),
        ],
    )(batch2, h, cW1, cb1, cW2, cb2)


def _gcn(h, edge_index, W, b, dinv):
    src = edge_index[0]
    dst = edge_index[1]
    hW = h @ W
    hs = hW * dinv[:, None]
    acc = jax.ops.segment_sum(hs[src], dst, num_segments=N)
    return (acc + hs) * dinv[:, None] + b


def kernel(x, edge_index, batch, emb, W1, b1, W2, b2, cW1, cb1, cW2, cb2):
    deg = jax.ops.segment_sum(
        jnp.ones((edge_index.shape[1],), jnp.float32), edge_index[1],
        num_segments=N) + 1.0
    dinv = jax.lax.rsqrt(deg)
    h = jnp.take(emb, x[:, 0], axis=0)
    h = jax.nn.relu(_gcn(h, edge_index, W1, b1, dinv))
    h = _gcn(h, edge_index, W2, b2, dinv)
    return _pool_mlp(batch, h, cW1, cb1, cW2, cb2)


# speedup vs baseline: 8.2157x; 2.7158x over previous
import functools

import jax
import jax.numpy as jnp
from jax import lax
from jax.experimental import pallas as pl
from jax.experimental.pallas import tpu as pltpu
from jax.experimental.pallas import tpu_sc as plsc

N = 50000
E = 800000
G = 128
T = 1000
D = 64
H = 128

NC = 2
NS = 16
L = 16

NROW = 53248
RB = 3328
NXROW = 512
EP = 819200
ER = EP // 128
ERT = ER // (NC * NS)
ERS = ER // NS
NCH = 4
CW = H // NCH

_mesh = plsc.VectorSubcoreMesh(core_axis_name="c", subcore_axis_name="s",
                               num_cores=NC, num_subcores=NS)


def _zero_vmem1d(ref, nelem):
    zero = jnp.zeros((L,), jnp.float32)

    def body(i, _):
        ref[pl.ds(i * L, L)] = zero
        return 0

    lax.fori_loop(0, nelem // L, body, 0)


def _zero_vmem2d(ref, nrows, ncols):
    zero = jnp.zeros((L,), jnp.float32)

    def body(i, _):
        for k in range(ncols // L):
            ref[i, pl.ds(k * L, L)] = zero
        return 0

    lax.fori_loop(0, nrows, body, 0)


def _sc_deg_embed(xp_hbm, dstp_hbm, emb_hbm, h0_hbm, hist_hbm,
                  idx_v, erows_v, hist_v, rows_v, sem):
    c = lax.axis_index("c")
    s = lax.axis_index("s")
    w = c * NS + s

    pltpu.sync_copy(dstp_hbm.at[pl.ds(w * ERT, ERT)], erows_v)
    _zero_vmem1d(hist_v, NROW)
    ones = jnp.full((L,), 1.0, jnp.float32)

    def erow(j, _):
        for k in range(128 // L):
            idx16 = erows_v[j, pl.ds(k * L, L)]
            plsc.addupdate_scatter(hist_v, [idx16], ones)
        return 0

    lax.fori_loop(0, ERT, erow, 0)
    pltpu.sync_copy(hist_v, hist_hbm.at[pl.ds(w * NROW, NROW)])

    nb = NXROW // (NC * NS)
    pltpu.sync_copy(xp_hbm.at[pl.ds(w * nb, nb)], idx_v)

    def nrow(j, _):
        pltpu.async_copy(emb_hbm.at[idx_v.at[j]], rows_v, sem).wait()
        pltpu.sync_copy(rows_v, h0_hbm.at[pl.ds((w * nb + j) * 128, 128)])
        return 0

    lax.fori_loop(0, nb, nrow, 0)


def _call_A(xp, dstp, emb):
    return pl.kernel(
        _sc_deg_embed,
        out_type=(
            jax.ShapeDtypeStruct((NXROW * 128, D), jnp.float32),
            jax.ShapeDtypeStruct((NC * NS * NROW,), jnp.float32),
        ),
        mesh=_mesh,
        scratch_types=[
            pltpu.VMEM((NXROW // (NC * NS), 128), jnp.int32),
            pltpu.VMEM((ERT, 128), jnp.int32),
            pltpu.VMEM((NROW,), jnp.float32),
            pltpu.VMEM((128, D), jnp.float32),
            pltpu.SemaphoreType.DMA,
        ],
        compiler_params=pltpu.CompilerParams(needs_layout_passes=False, use_tc_tiling_on_sc=False),
    )(xp, dstp, emb)


IB = 8
NBLK = ERS // IB


def _sc_mp(hs_hbm, src4_hbm, dstp_hbm, acc_hbm, sb0, sb1, db0, db1,
           buf0, buf1, zb_v, isem0, isem1, gsem0, gsem1, acc_sp):
    c = lax.axis_index("c")
    s = lax.axis_index("s")
    _zero_vmem2d(zb_v, 128, CW)
    sbs = (sb0, sb1)
    dbs = (db0, db1)
    bufs = (buf0, buf1)
    isems = (isem0, isem1)
    gsems = (gsem0, gsem1)

    for k in range(2):
        ch = 2 * c + k

        def load_ids(bb, par):
            base = ch * ER + s * ERS + bb * IB
            pltpu.async_copy(src4_hbm.at[pl.ds(base, IB)], sbs[par],
                             isems[par])
            pltpu.async_copy(dstp_hbm.at[pl.ds(s * ERS + bb * IB, IB)],
                             dbs[par], isems[par])

        def wait_ids(bb, par):
            base = ch * ER + s * ERS + bb * IB
            pltpu.make_async_copy(src4_hbm.at[pl.ds(base, IB)], sbs[par],
                                  isems[par]).wait()
            pltpu.make_async_copy(dstp_hbm.at[pl.ds(s * ERS + bb * IB, IB)],
                                  dbs[par], isems[par]).wait()

        def zrow(i, _):
            pltpu.sync_copy(zb_v, acc_sp.at[pl.ds(s * RB + i * 128, 128)])
            return 0
        lax.fori_loop(0, RB // 128, zrow, 0)
        plsc.subcore_barrier()

        load_ids(0, 0)

        def pair(bp, _):
            for q in range(2):
                bb = bp * 2 + q
                wait_ids(bb, q)

                @pl.when(bb + 1 < NBLK)
                def _pref():
                    load_ids(bb + 1, 1 - q)

                pltpu.async_copy(hs_hbm.at[sbs[q].at[0]], bufs[0], gsems[0])
                for r in range(IB):
                    pltpu.make_async_copy(hs_hbm.at[sbs[q].at[r]],
                                          bufs[r % 2], gsems[r % 2]).wait()
                    if r + 1 < IB:
                        pltpu.async_copy(hs_hbm.at[sbs[q].at[r + 1]],
                                         bufs[(r + 1) % 2],
                                         gsems[(r + 1) % 2])
                    pltpu.sync_copy(bufs[r % 2], acc_sp.at[dbs[q].at[r]],
                                    add=True)
            return 0

        lax.fori_loop(0, NBLK // 2, pair, 0)
        plsc.subcore_barrier()

        pltpu.sync_copy(acc_sp.at[pl.ds(s * RB, RB)],
                        acc_hbm.at[pl.ds(ch * NROW + s * RB, RB)])


def _call_mp(hs_flat, src4, dstp):
    return pl.kernel(
        _sc_mp,
        out_type=jax.ShapeDtypeStruct((NCH * NROW, CW), jnp.float32),
        mesh=_mesh,
        scratch_types=[
            pltpu.VMEM((IB, 128), jnp.int32),
            pltpu.VMEM((IB, 128), jnp.int32),
            pltpu.VMEM((IB, 128), jnp.int32),
            pltpu.VMEM((IB, 128), jnp.int32),
            pltpu.VMEM((128, CW), jnp.float32),
            pltpu.VMEM((128, CW), jnp.float32),
            pltpu.VMEM((128, CW), jnp.float32),
            pltpu.SemaphoreType.DMA,
            pltpu.SemaphoreType.DMA,
            pltpu.SemaphoreType.DMA,
            pltpu.SemaphoreType.DMA,
            pltpu.VMEM_SHARED((NROW, CW), jnp.float32),
        ],
        compiler_params=pltpu.CompilerParams(needs_layout_passes=False, use_tc_tiling_on_sc=False),
    )(hs_flat, src4, dstp)


def _tc_prep(hist_ref, h0_ref, W1_ref, dinv_ref, hs1_ref):
    deg = jnp.sum(hist_ref[...], axis=0) + 1.0
    dinv = lax.rsqrt(deg)
    dinv_ref[0, 0, :] = dinv
    hs = jnp.dot(h0_ref[...], W1_ref[...],
                 preferred_element_type=jnp.float32) * dinv[:, None]
    for cch in range(NCH):
        hs1_ref[cch] = hs[:, cch * CW:(cch + 1) * CW]


def _call_B(hist, h0, W1):
    nblk = NROW // RB
    return pl.pallas_call(
        _tc_prep,
        grid=(nblk,),
        in_specs=[
            pl.BlockSpec((NC * NS, RB), lambda i: (0, i)),
            pl.BlockSpec((RB, D), lambda i: (i, 0)),
            pl.BlockSpec((D, H), lambda i: (0, 0)),
        ],
        out_specs=[
            pl.BlockSpec((1, 1, RB), lambda i: (i, 0, 0)),
            pl.BlockSpec((NCH, RB, CW), lambda i: (0, i, 0)),
        ],
        out_shape=[
            jax.ShapeDtypeStruct((NROW // RB, 1, RB), jnp.float32),
            jax.ShapeDtypeStruct((NCH, NROW, CW), jnp.float32),
        ],
    )(hist, h0, W1)


def _tc_mid(acc_ref, hs_ref, dinv_ref, b1_ref, W2_ref, hs2_ref):
    full = jnp.concatenate([acc_ref[cch] + hs_ref[cch] for cch in range(NCH)],
                           axis=1)
    dinv = dinv_ref[0, 0, :]
    h1 = jnp.maximum(full * dinv[:, None] + b1_ref[0, :][None, :], 0.0)
    hs2 = jnp.dot(h1, W2_ref[...],
                  preferred_element_type=jnp.float32) * dinv[:, None]
    for cch in range(NCH):
        hs2_ref[cch] = hs2[:, cch * CW:(cch + 1) * CW]


def _call_D(acc1, hs1, dinv2d, b1, W2):
    nblk = NROW // RB
    return pl.pallas_call(
        _tc_mid,
        grid=(nblk,),
        in_specs=[
            pl.BlockSpec((NCH, RB, CW), lambda i: (0, i, 0)),
            pl.BlockSpec((NCH, RB, CW), lambda i: (0, i, 0)),
            pl.BlockSpec((1, 1, RB), lambda i: (i, 0, 0)),
            pl.BlockSpec((1, H), lambda i: (0, 0)),
            pl.BlockSpec((H, H), lambda i: (0, 0)),
        ],
        out_specs=pl.BlockSpec((NCH, RB, CW), lambda i: (0, i, 0)),
        out_shape=jax.ShapeDtypeStruct((NCH, NROW, CW), jnp.float32),
    )(acc1, hs1, dinv2d, b1, W2)


def _tc_final(acc_ref, hs_ref, dinv_ref, b2_ref, batch_ref, cW1_ref, cb1_ref,
              cW2_ref, cb2_ref, out_ref, accg_ref, cnt_ref):
    i = pl.program_id(0)
    nsteps = pl.num_programs(0)

    @pl.when(i == 0)
    def _init():
        accg_ref[...] = jnp.zeros_like(accg_ref)
        cnt_ref[...] = jnp.zeros_like(cnt_ref)

    full = jnp.concatenate([acc_ref[cch] + hs_ref[cch] for cch in range(NCH)],
                           axis=1)
    dinv = dinv_ref[0, 0, :]
    out2 = full * dinv[:, None] + b2_ref[0, :][None, :]
    b = batch_ref[0, 0, :]
    seg = lax.broadcasted_iota(jnp.int32, (G, 1), 0)
    onehot = jnp.where(b[None, :] == seg, 1.0, 0.0)
    accg_ref[...] += jnp.dot(onehot, out2, preferred_element_type=jnp.float32)
    cnt_ref[...] += jnp.sum(onehot, axis=1, keepdims=True)

    @pl.when(i == nsteps - 1)
    def _fin():
        g = accg_ref[...] / jnp.maximum(cnt_ref[...], 1.0)
        z = jnp.maximum(
            jnp.dot(g, cW1_ref[...], preferred_element_type=jnp.float32)
            + cb1_ref[0, :][None, :], 0.0)
        out_ref[...] = (jnp.dot(z, cW2_ref[...],
                                preferred_element_type=jnp.float32)
                        + cb2_ref[0, :][None, :])


def _call_F(acc2, hs2, dinv2d, b2, batchp, cW1, cb1, cW2, cb2):
    nblk = NROW // RB
    return pl.pallas_call(
        _tc_final,
        grid=(nblk,),
        in_specs=[
            pl.BlockSpec((NCH, RB, CW), lambda i: (0, i, 0)),
            pl.BlockSpec((NCH, RB, CW), lambda i: (0, i, 0)),
            pl.BlockSpec((1, 1, RB), lambda i: (i, 0, 0)),
            pl.BlockSpec((1, H), lambda i: (0, 0)),
            pl.BlockSpec((1, 1, RB), lambda i: (i, 0, 0)),
            pl.BlockSpec((H, H // 2), lambda i: (0, 0)),
            pl.BlockSpec((1, H // 2), lambda i: (0, 0)),
            pl.BlockSpec((H // 2, 1), lambda i: (0, 0)),
            pl.BlockSpec((1, 1), lambda i: (0, 0)),
        ],
        out_specs=pl.BlockSpec((G, 1), lambda i: (0, 0)),
        out_shape=jax.ShapeDtypeStruct((G, 1), jnp.float32),
        scratch_shapes=[
            pltpu.VMEM((G, H), jnp.float32),
            pltpu.VMEM((G, 1), jnp.float32),
        ],
    )(acc2, hs2, dinv2d, b2, batchp, cW1, cb1, cW2, cb2)


def kernel(x, edge_index, batch, emb, W1, b1, W2, b2, cW1, cb1, cW2, cb2):
    x = x.astype(jnp.int32)
    edge_index = edge_index.astype(jnp.int32)
    batch = batch.astype(jnp.int32)

    xp = jnp.pad(x[:, 0], (0, NXROW * 128 - N)).reshape(NXROW, 128)
    pad_e = EP - E
    srcp = jnp.concatenate(
        [edge_index[0], jnp.zeros((pad_e,), jnp.int32)]).reshape(ER, 128)
    trash = N + (jnp.arange(pad_e, dtype=jnp.int32) % (NROW - N - 128))
    dstp = jnp.concatenate([edge_index[1], trash]).reshape(ER, 128)
    batchp = jnp.pad(batch, (0, NROW - N),
                     constant_values=2 * G).reshape(NROW // RB, 1, RB)

    h0, hist = _call_A(xp, dstp, emb)
    dinv2d, hs1 = _call_B(hist.reshape(NC * NS, NROW), h0[:NROW], W1)
    src4 = (srcp[None, :, :]
            + (jnp.arange(NCH, dtype=jnp.int32) * NROW)[:, None, None]
            ).reshape(NCH * ER, 128)
    acc1 = _call_mp(hs1.reshape(NCH * NROW, CW), src4, dstp)
    hs2 = _call_D(acc1.reshape(NCH, NROW, CW), hs1, dinv2d,
                  b1.reshape(1, H), W2)
    acc2 = _call_mp(hs2.reshape(NCH * NROW, CW), src4, dstp)
    return _call_F(acc2.reshape(NCH, NROW, CW), hs2, dinv2d,
                   b2.reshape(1, H), batchp, cW1, cb1.reshape(1, H // 2),
                   cW2, cb2.reshape(1, 1))


# speedup vs baseline: 9.0167x; 1.0975x over previous
import functools

import jax
import jax.numpy as jnp
from jax import lax
from jax.experimental import pallas as pl
from jax.experimental.pallas import tpu as pltpu
from jax.experimental.pallas import tpu_sc as plsc

N = 50000
E = 800000
G = 128
T = 1000
D = 64
H = 128

NC = 2
NS = 16
L = 16

NROW = 53248
RB = 3328
NXROW = 512
EP = 819200
ER = EP // 128
ERT = 100
ERS = ER // NS
NCH = 4
CW = H // NCH

_mesh = plsc.VectorSubcoreMesh(core_axis_name="c", subcore_axis_name="s",
                               num_cores=NC, num_subcores=NS)


def _zero_vmem1d(ref, nelem):
    zero = jnp.zeros((L,), jnp.float32)

    def body(i, _):
        ref[pl.ds(i * L, L)] = zero
        return 0

    lax.fori_loop(0, nelem // L, body, 0)


def _zero_vmem2d(ref, nrows, ncols):
    zero = jnp.zeros((L,), jnp.float32)

    def body(i, _):
        for k in range(ncols // L):
            ref[i, pl.ds(k * L, L)] = zero
        return 0

    lax.fori_loop(0, nrows, body, 0)


def _sc_deg_embed(xp_hbm, dstp_hbm, emb_hbm, h0_hbm, hist_hbm,
                  idx_v, erows_v, hist_v, rows_v, sem):
    c = lax.axis_index("c")
    s = lax.axis_index("s")
    w = c * NS + s

    pltpu.sync_copy(dstp_hbm.at[pl.ds(w * ERT, ERT)], erows_v)
    _zero_vmem1d(hist_v, NROW)
    ones = jnp.full((L,), 1.0, jnp.float32)

    def erow(j, _):
        for k in range(256 // L):
            idx16 = erows_v[j, pl.ds(k * L, L)]
            plsc.addupdate_scatter(hist_v, [idx16], ones)
        return 0

    lax.fori_loop(0, ERT, erow, 0)
    pltpu.sync_copy(hist_v, hist_hbm.at[pl.ds(w * NROW, NROW)])

    nb = NXROW // (NC * NS)
    pltpu.sync_copy(xp_hbm.at[pl.ds(w * nb, nb)], idx_v)

    def nrow(j, _):
        pltpu.async_copy(emb_hbm.at[idx_v.at[j]], rows_v, sem).wait()
        pltpu.sync_copy(rows_v, h0_hbm.at[pl.ds((w * nb + j) * 128, 128)])
        return 0

    lax.fori_loop(0, nb, nrow, 0)


def _call_A(xp, dstp, emb):
    return pl.kernel(
        _sc_deg_embed,
        out_type=(
            jax.ShapeDtypeStruct((NXROW * 128, D), jnp.float32),
            jax.ShapeDtypeStruct((NC * NS * NROW,), jnp.float32),
        ),
        mesh=_mesh,
        scratch_types=[
            pltpu.VMEM((NXROW // (NC * NS), 128), jnp.int32),
            pltpu.VMEM((ERT, 256), jnp.int32),
            pltpu.VMEM((NROW,), jnp.float32),
            pltpu.VMEM((128, D), jnp.float32),
            pltpu.SemaphoreType.DMA,
        ],
        compiler_params=pltpu.CompilerParams(needs_layout_passes=False, use_tc_tiling_on_sc=False),
    )(xp, dstp, emb)


IDW = 256
IB = 10
ERS2 = 200
NBLK = ERS2 // IB
ERX = ER + 128


def _sc_mp(hs_hbm, srcp_hbm, dstp_hbm, acc_hbm,
           sb, db, buf0, buf1, zb_v, isem, gsem0, gsem1, acc_sp):
    c = lax.axis_index("c")
    s = lax.axis_index("s")
    _zero_vmem2d(zb_v, 32, CW)
    bufs = (buf0, buf1)
    gsems = (gsem0, gsem1)

    for k in range(2):
        ch = 2 * c + k
        off = ch * NROW

        def load_offset_ids(bb):
            base = s * ERS2 + bb * IB
            pltpu.async_copy(srcp_hbm.at[pl.ds(base, IB)], sb, isem)
            pltpu.async_copy(dstp_hbm.at[pl.ds(base, IB)], db, isem)
            pltpu.make_async_copy(srcp_hbm.at[pl.ds(base, IB)], sb,
                                  isem).wait()
            pltpu.make_async_copy(dstp_hbm.at[pl.ds(base, IB)], db,
                                  isem).wait()
            offv = jnp.full((L,), 1, jnp.int32) * off
            for r8 in range(IB):
                for k2 in range(IDW // L):
                    sb[r8, pl.ds(k2 * L, L)] = (
                        sb[r8, pl.ds(k2 * L, L)] + offv)

        def gather(r2, vb):
            pltpu.async_copy(hs_hbm.at[sb.at[r2]], bufs[vb], gsems[vb])

        def gather_wait(r2, vb):
            pltpu.make_async_copy(hs_hbm.at[sb.at[r2]], bufs[vb],
                                  gsems[vb]).wait()

        def scat(r2, vb):
            pltpu.sync_copy(bufs[vb], acc_sp.at[db.at[r2]], add=True)

        def zrow(i, _):
            pltpu.sync_copy(zb_v, acc_sp.at[pl.ds(s * RB + i * 32, 32)])
            return 0
        lax.fori_loop(0, RB // 32, zrow, 0)
        plsc.subcore_barrier()

        def block(bb, _):
            load_offset_ids(bb)
            gather(0, 0)
            for r2 in range(IB):
                gather_wait(r2, r2 % 2)
                if r2 + 1 < IB:
                    gather(r2 + 1, (r2 + 1) % 2)
                scat(r2, r2 % 2)
            return 0

        lax.fori_loop(0, NBLK, block, 0)
        plsc.subcore_barrier()

        pltpu.sync_copy(acc_sp.at[pl.ds(s * RB, RB)],
                        acc_hbm.at[pl.ds(ch * NROW + s * RB, RB)])


def _call_mp(hs_flat, srcp, dstp):
    return pl.kernel(
        _sc_mp,
        out_type=jax.ShapeDtypeStruct((NCH * NROW, CW), jnp.float32),
        mesh=_mesh,
        scratch_types=[
            pltpu.VMEM((IB, IDW), jnp.int32),
            pltpu.VMEM((IB, IDW), jnp.int32),
            pltpu.VMEM((IDW, CW), jnp.float32),
            pltpu.VMEM((IDW, CW), jnp.float32),
            pltpu.VMEM((32, CW), jnp.float32),
            pltpu.SemaphoreType.DMA,
            pltpu.SemaphoreType.DMA,
            pltpu.SemaphoreType.DMA,
            pltpu.VMEM_SHARED((NROW, CW), jnp.float32),
        ],
        compiler_params=pltpu.CompilerParams(needs_layout_passes=False, use_tc_tiling_on_sc=False),
    )(hs_flat, srcp, dstp)


def _tc_prep(hist_ref, h0_ref, W1_ref, dinv_ref, hs1_ref):
    deg = jnp.sum(hist_ref[...], axis=0) + 1.0
    dinv = lax.rsqrt(deg)
    dinv_ref[0, 0, :] = dinv
    hs = jnp.dot(h0_ref[...], W1_ref[...],
                 preferred_element_type=jnp.float32) * dinv[:, None]
    for cch in range(NCH):
        hs1_ref[cch] = hs[:, cch * CW:(cch + 1) * CW]


def _call_B(hist, h0, W1):
    nblk = NROW // RB
    return pl.pallas_call(
        _tc_prep,
        grid=(nblk,),
        in_specs=[
            pl.BlockSpec((NC * NS, RB), lambda i: (0, i)),
            pl.BlockSpec((RB, D), lambda i: (i, 0)),
            pl.BlockSpec((D, H), lambda i: (0, 0)),
        ],
        out_specs=[
            pl.BlockSpec((1, 1, RB), lambda i: (i, 0, 0)),
            pl.BlockSpec((NCH, RB, CW), lambda i: (0, i, 0)),
        ],
        out_shape=[
            jax.ShapeDtypeStruct((NROW // RB, 1, RB), jnp.float32),
            jax.ShapeDtypeStruct((NCH, NROW, CW), jnp.float32),
        ],
    )(hist, h0, W1)


def _tc_mid(acc_ref, hs_ref, dinv_ref, b1_ref, W2_ref, hs2_ref):
    full = jnp.concatenate([acc_ref[cch] + hs_ref[cch] for cch in range(NCH)],
                           axis=1)
    dinv = dinv_ref[0, 0, :]
    h1 = jnp.maximum(full * dinv[:, None] + b1_ref[0, :][None, :], 0.0)
    hs2 = jnp.dot(h1, W2_ref[...],
                  preferred_element_type=jnp.float32) * dinv[:, None]
    for cch in range(NCH):
        hs2_ref[cch] = hs2[:, cch * CW:(cch + 1) * CW]


def _call_D(acc1, hs1, dinv2d, b1, W2):
    nblk = NROW // RB
    return pl.pallas_call(
        _tc_mid,
        grid=(nblk,),
        in_specs=[
            pl.BlockSpec((NCH, RB, CW), lambda i: (0, i, 0)),
            pl.BlockSpec((NCH, RB, CW), lambda i: (0, i, 0)),
            pl.BlockSpec((1, 1, RB), lambda i: (i, 0, 0)),
            pl.BlockSpec((1, H), lambda i: (0, 0)),
            pl.BlockSpec((H, H), lambda i: (0, 0)),
        ],
        out_specs=pl.BlockSpec((NCH, RB, CW), lambda i: (0, i, 0)),
        out_shape=jax.ShapeDtypeStruct((NCH, NROW, CW), jnp.float32),
    )(acc1, hs1, dinv2d, b1, W2)


def _tc_final(acc_ref, hs_ref, dinv_ref, b2_ref, batch_ref, cW1_ref, cb1_ref,
              cW2_ref, cb2_ref, out_ref, accg_ref, cnt_ref):
    i = pl.program_id(0)
    nsteps = pl.num_programs(0)

    @pl.when(i == 0)
    def _init():
        accg_ref[...] = jnp.zeros_like(accg_ref)
        cnt_ref[...] = jnp.zeros_like(cnt_ref)

    full = jnp.concatenate([acc_ref[cch] + hs_ref[cch] for cch in range(NCH)],
                           axis=1)
    dinv = dinv_ref[0, 0, :]
    out2 = full * dinv[:, None] + b2_ref[0, :][None, :]
    b = batch_ref[0, 0, :]
    seg = lax.broadcasted_iota(jnp.int32, (G, 1), 0)
    onehot = jnp.where(b[None, :] == seg, 1.0, 0.0)
    accg_ref[...] += jnp.dot(onehot, out2, preferred_element_type=jnp.float32)
    cnt_ref[...] += jnp.sum(onehot, axis=1, keepdims=True)

    @pl.when(i == nsteps - 1)
    def _fin():
        g = accg_ref[...] / jnp.maximum(cnt_ref[...], 1.0)
        z = jnp.maximum(
            jnp.dot(g, cW1_ref[...], preferred_element_type=jnp.float32)
            + cb1_ref[0, :][None, :], 0.0)
        out_ref[...] = (jnp.dot(z, cW2_ref[...],
                                preferred_element_type=jnp.float32)
                        + cb2_ref[0, :][None, :])


def _call_F(acc2, hs2, dinv2d, b2, batchp, cW1, cb1, cW2, cb2):
    nblk = NROW // RB
    return pl.pallas_call(
        _tc_final,
        grid=(nblk,),
        in_specs=[
            pl.BlockSpec((NCH, RB, CW), lambda i: (0, i, 0)),
            pl.BlockSpec((NCH, RB, CW), lambda i: (0, i, 0)),
            pl.BlockSpec((1, 1, RB), lambda i: (i, 0, 0)),
            pl.BlockSpec((1, H), lambda i: (0, 0)),
            pl.BlockSpec((1, 1, RB), lambda i: (i, 0, 0)),
            pl.BlockSpec((H, H // 2), lambda i: (0, 0)),
            pl.BlockSpec((1, H // 2), lambda i: (0, 0)),
            pl.BlockSpec((H // 2, 1), lambda i: (0, 0)),
            pl.BlockSpec((1, 1), lambda i: (0, 0)),
        ],
        out_specs=pl.BlockSpec((G, 1), lambda i: (0, 0)),
        out_shape=jax.ShapeDtypeStruct((G, 1), jnp.float32),
        scratch_shapes=[
            pltpu.VMEM((G, H), jnp.float32),
            pltpu.VMEM((G, 1), jnp.float32),
        ],
    )(acc2, hs2, dinv2d, b2, batchp, cW1, cb1, cW2, cb2)


def kernel(x, edge_index, batch, emb, W1, b1, W2, b2, cW1, cb1, cW2, cb2):
    x = x.astype(jnp.int32)
    edge_index = edge_index.astype(jnp.int32)
    batch = batch.astype(jnp.int32)

    xp = jnp.pad(x[:, 0], (0, NXROW * 128 - N)).reshape(NXROW, 128)
    pad_e = EP - E
    srcp = jnp.concatenate(
        [edge_index[0],
         jnp.zeros((EP + 16384 - E,), jnp.int32)]).reshape(ERX // 2, 256)
    trash = N + (jnp.arange(EP - E, dtype=jnp.int32) % (NROW - N - 128))
    dstp = jnp.concatenate(
        [edge_index[1], trash,
         jnp.zeros((16384,), jnp.int32)]).reshape(ERX // 2, 256)
    batchp = jnp.pad(batch, (0, NROW - N),
                     constant_values=2 * G).reshape(NROW // RB, 1, RB)

    h0, hist = _call_A(xp, dstp, emb)
    dinv2d, hs1 = _call_B(hist.reshape(NC * NS, NROW), h0[:NROW], W1)
    acc1 = _call_mp(hs1.reshape(NCH * NROW, CW), srcp, dstp)
    hs2 = _call_D(acc1.reshape(NCH, NROW, CW), hs1, dinv2d,
                  b1.reshape(1, H), W2)
    acc2 = _call_mp(hs2.reshape(NCH * NROW, CW), srcp, dstp)
    return _call_F(acc2.reshape(NCH, NROW, CW), hs2, dinv2d,
                   b2.reshape(1, H), batchp, cW1, cb1.reshape(1, H // 2),
                   cW2, cb2.reshape(1, 1))


# speedup vs baseline: 9.0953x; 1.0087x over previous
import functools

import jax
import jax.numpy as jnp
from jax import lax
from jax.experimental import pallas as pl
from jax.experimental.pallas import tpu as pltpu
from jax.experimental.pallas import tpu_sc as plsc

N = 50000
E = 800000
G = 128
T = 1000
D = 64
H = 128

NC = 2
NS = 16
L = 16

NROW = 53248
RB = 3328
NXROW = 512
EP = 819200
ER = EP // 128
ERT = 100
ERS = ER // NS
NCH = 4
CW = H // NCH

_mesh = plsc.VectorSubcoreMesh(core_axis_name="c", subcore_axis_name="s",
                               num_cores=NC, num_subcores=NS)


def _zero_vmem1d(ref, nelem):
    zero = jnp.zeros((L,), jnp.float32)

    def body(i, _):
        ref[pl.ds(i * L, L)] = zero
        return 0

    lax.fori_loop(0, nelem // L, body, 0)


def _zero_vmem2d(ref, nrows, ncols):
    zero = jnp.zeros((L,), jnp.float32)

    def body(i, _):
        for k in range(ncols // L):
            ref[i, pl.ds(k * L, L)] = zero
        return 0

    lax.fori_loop(0, nrows, body, 0)


def _sc_deg_embed(xp_hbm, dstp_hbm, emb_hbm, h0_hbm, hist_hbm,
                  idx_v, erows_v, hist_v, rows_v, sem):
    c = lax.axis_index("c")
    s = lax.axis_index("s")
    w = c * NS + s

    pltpu.sync_copy(dstp_hbm.at[pl.ds(w * ERT, ERT)], erows_v)
    _zero_vmem1d(hist_v, NROW)
    ones = jnp.full((L,), 1.0, jnp.float32)

    def erow(j, _):
        for k in range(256 // L):
            idx16 = erows_v[j, pl.ds(k * L, L)]
            plsc.addupdate_scatter(hist_v, [idx16], ones)
        return 0

    lax.fori_loop(0, ERT, erow, 0)
    pltpu.sync_copy(hist_v, hist_hbm.at[pl.ds(w * NROW, NROW)])

    nb = NXROW // (NC * NS)
    pltpu.sync_copy(xp_hbm.at[pl.ds(w * nb, nb)], idx_v)

    def nrow(j, _):
        pltpu.async_copy(emb_hbm.at[idx_v.at[j]], rows_v, sem).wait()
        pltpu.sync_copy(rows_v, h0_hbm.at[pl.ds((w * nb + j) * 128, 128)])
        return 0

    lax.fori_loop(0, nb, nrow, 0)


def _call_A(xp, dstp, emb):
    return pl.kernel(
        _sc_deg_embed,
        out_type=(
            jax.ShapeDtypeStruct((NXROW * 128, D), jnp.float32),
            jax.ShapeDtypeStruct((NC * NS * NROW,), jnp.float32),
        ),
        mesh=_mesh,
        scratch_types=[
            pltpu.VMEM((NXROW // (NC * NS), 128), jnp.int32),
            pltpu.VMEM((ERT, 256), jnp.int32),
            pltpu.VMEM((NROW,), jnp.float32),
            pltpu.VMEM((128, D), jnp.float32),
            pltpu.SemaphoreType.DMA,
        ],
        compiler_params=pltpu.CompilerParams(needs_layout_passes=False, use_tc_tiling_on_sc=False),
    )(xp, dstp, emb)


IDW = 256
IB = 10
ERS2 = 200
NBLK = ERS2 // IB
ERX = ER + 128


def _sc_mp(hs_hbm, srcp_hbm, dstp_hbm, acc_hbm,
           sb, db, buf0, buf1, zb_v, isem, gsem0, gsem1, zsem, acc_sp):
    c = lax.axis_index("c")
    s = lax.axis_index("s")
    _zero_vmem2d(zb_v, 32, CW)
    bufs = (buf0, buf1)
    gsems = (gsem0, gsem1)

    for k in range(2):
        ch = 2 * c + k
        off = ch * NROW

        def load_offset_ids(bb):
            base = s * ERS2 + bb * IB
            pltpu.async_copy(srcp_hbm.at[pl.ds(base, IB)], sb, isem)
            pltpu.async_copy(dstp_hbm.at[pl.ds(base, IB)], db, isem)
            pltpu.make_async_copy(srcp_hbm.at[pl.ds(base, IB)], sb,
                                  isem).wait()
            pltpu.make_async_copy(dstp_hbm.at[pl.ds(base, IB)], db,
                                  isem).wait()
            offv = jnp.full((L,), 1, jnp.int32) * off
            for r8 in range(IB):
                for k2 in range(IDW // L):
                    sb[r8, pl.ds(k2 * L, L)] = (
                        sb[r8, pl.ds(k2 * L, L)] + offv)

        def gather(r2, vb):
            pltpu.async_copy(hs_hbm.at[sb.at[r2]], bufs[vb], gsems[vb])

        def gather_wait(r2, vb):
            pltpu.make_async_copy(hs_hbm.at[sb.at[r2]], bufs[vb],
                                  gsems[vb]).wait()

        def scat(r2, vb):
            pltpu.sync_copy(bufs[vb], acc_sp.at[db.at[r2]], add=True)

        def zrow(i, _):
            pltpu.async_copy(zb_v, acc_sp.at[pl.ds(s * RB + i * 32, 32)],
                             zsem)
            return 0
        lax.fori_loop(0, RB // 32, zrow, 0)

        def zdrain(i, _):
            pltpu.make_async_copy(zb_v, acc_sp.at[pl.ds(s * RB + i * 32, 32)],
                                  zsem).wait()
            return 0
        lax.fori_loop(0, RB // 32, zdrain, 0)
        plsc.subcore_barrier()

        def block(bb, _):
            load_offset_ids(bb)
            gather(0, 0)
            for r2 in range(IB):
                gather_wait(r2, r2 % 2)
                if r2 + 1 < IB:
                    gather(r2 + 1, (r2 + 1) % 2)
                scat(r2, r2 % 2)
            return 0

        lax.fori_loop(0, NBLK, block, 0)
        plsc.subcore_barrier()

        pltpu.sync_copy(acc_sp.at[pl.ds(s * RB, RB)],
                        acc_hbm.at[pl.ds(ch * NROW + s * RB, RB)])


def _call_mp(hs_flat, srcp, dstp):
    return pl.kernel(
        _sc_mp,
        out_type=jax.ShapeDtypeStruct((NCH * NROW, CW), jnp.float32),
        mesh=_mesh,
        scratch_types=[
            pltpu.VMEM((IB, IDW), jnp.int32),
            pltpu.VMEM((IB, IDW), jnp.int32),
            pltpu.VMEM((IDW, CW), jnp.float32),
            pltpu.VMEM((IDW, CW), jnp.float32),
            pltpu.VMEM((32, CW), jnp.float32),
            pltpu.SemaphoreType.DMA,
            pltpu.SemaphoreType.DMA,
            pltpu.SemaphoreType.DMA,
            pltpu.SemaphoreType.DMA,
            pltpu.VMEM_SHARED((NROW, CW), jnp.float32),
        ],
        compiler_params=pltpu.CompilerParams(needs_layout_passes=False, use_tc_tiling_on_sc=False),
    )(hs_flat, srcp, dstp)


def _tc_prep(hist_ref, h0_ref, W1_ref, dinv_ref, hs1_ref):
    deg = jnp.sum(hist_ref[...], axis=0) + 1.0
    dinv = lax.rsqrt(deg)
    dinv_ref[0, 0, :] = dinv
    hs = jnp.dot(h0_ref[...], W1_ref[...],
                 preferred_element_type=jnp.float32) * dinv[:, None]
    for cch in range(NCH):
        hs1_ref[cch] = hs[:, cch * CW:(cch + 1) * CW]


def _call_B(hist, h0, W1):
    nblk = NROW // RB
    return pl.pallas_call(
        _tc_prep,
        grid=(nblk,),
        in_specs=[
            pl.BlockSpec((NC * NS, RB), lambda i: (0, i)),
            pl.BlockSpec((RB, D), lambda i: (i, 0)),
            pl.BlockSpec((D, H), lambda i: (0, 0)),
        ],
        out_specs=[
            pl.BlockSpec((1, 1, RB), lambda i: (i, 0, 0)),
            pl.BlockSpec((NCH, RB, CW), lambda i: (0, i, 0)),
        ],
        out_shape=[
            jax.ShapeDtypeStruct((NROW // RB, 1, RB), jnp.float32),
            jax.ShapeDtypeStruct((NCH, NROW, CW), jnp.float32),
        ],
    )(hist, h0, W1)


def _tc_mid(acc_ref, hs_ref, dinv_ref, b1_ref, W2_ref, hs2_ref):
    full = jnp.concatenate([acc_ref[cch] + hs_ref[cch] for cch in range(NCH)],
                           axis=1)
    dinv = dinv_ref[0, 0, :]
    h1 = jnp.maximum(full * dinv[:, None] + b1_ref[0, :][None, :], 0.0)
    hs2 = jnp.dot(h1, W2_ref[...],
                  preferred_element_type=jnp.float32) * dinv[:, None]
    for cch in range(NCH):
        hs2_ref[cch] = hs2[:, cch * CW:(cch + 1) * CW]


def _call_D(acc1, hs1, dinv2d, b1, W2):
    nblk = NROW // RB
    return pl.pallas_call(
        _tc_mid,
        grid=(nblk,),
        in_specs=[
            pl.BlockSpec((NCH, RB, CW), lambda i: (0, i, 0)),
            pl.BlockSpec((NCH, RB, CW), lambda i: (0, i, 0)),
            pl.BlockSpec((1, 1, RB), lambda i: (i, 0, 0)),
            pl.BlockSpec((1, H), lambda i: (0, 0)),
            pl.BlockSpec((H, H), lambda i: (0, 0)),
        ],
        out_specs=pl.BlockSpec((NCH, RB, CW), lambda i: (0, i, 0)),
        out_shape=jax.ShapeDtypeStruct((NCH, NROW, CW), jnp.float32),
    )(acc1, hs1, dinv2d, b1, W2)


def _tc_final(acc_ref, hs_ref, dinv_ref, b2_ref, batch_ref, cW1_ref, cb1_ref,
              cW2_ref, cb2_ref, out_ref, accg_ref, cnt_ref):
    i = pl.program_id(0)
    nsteps = pl.num_programs(0)

    @pl.when(i == 0)
    def _init():
        accg_ref[...] = jnp.zeros_like(accg_ref)
        cnt_ref[...] = jnp.zeros_like(cnt_ref)

    full = jnp.concatenate([acc_ref[cch] + hs_ref[cch] for cch in range(NCH)],
                           axis=1)
    dinv = dinv_ref[0, 0, :]
    out2 = full * dinv[:, None] + b2_ref[0, :][None, :]
    b = batch_ref[0, 0, :]
    seg = lax.broadcasted_iota(jnp.int32, (G, 1), 0)
    onehot = jnp.where(b[None, :] == seg, 1.0, 0.0)
    accg_ref[...] += jnp.dot(onehot, out2, preferred_element_type=jnp.float32)
    cnt_ref[...] += jnp.sum(onehot, axis=1, keepdims=True)

    @pl.when(i == nsteps - 1)
    def _fin():
        g = accg_ref[...] / jnp.maximum(cnt_ref[...], 1.0)
        z = jnp.maximum(
            jnp.dot(g, cW1_ref[...], preferred_element_type=jnp.float32)
            + cb1_ref[0, :][None, :], 0.0)
        out_ref[...] = (jnp.dot(z, cW2_ref[...],
                                preferred_element_type=jnp.float32)
                        + cb2_ref[0, :][None, :])


def _call_F(acc2, hs2, dinv2d, b2, batchp, cW1, cb1, cW2, cb2):
    nblk = NROW // RB
    return pl.pallas_call(
        _tc_final,
        grid=(nblk,),
        in_specs=[
            pl.BlockSpec((NCH, RB, CW), lambda i: (0, i, 0)),
            pl.BlockSpec((NCH, RB, CW), lambda i: (0, i, 0)),
            pl.BlockSpec((1, 1, RB), lambda i: (i, 0, 0)),
            pl.BlockSpec((1, H), lambda i: (0, 0)),
            pl.BlockSpec((1, 1, RB), lambda i: (i, 0, 0)),
            pl.BlockSpec((H, H // 2), lambda i: (0, 0)),
            pl.BlockSpec((1, H // 2), lambda i: (0, 0)),
            pl.BlockSpec((H // 2, 1), lambda i: (0, 0)),
            pl.BlockSpec((1, 1), lambda i: (0, 0)),
        ],
        out_specs=pl.BlockSpec((G, 1), lambda i: (0, 0)),
        out_shape=jax.ShapeDtypeStruct((G, 1), jnp.float32),
        scratch_shapes=[
            pltpu.VMEM((G, H), jnp.float32),
            pltpu.VMEM((G, 1), jnp.float32),
        ],
    )(acc2, hs2, dinv2d, b2, batchp, cW1, cb1, cW2, cb2)


def kernel(x, edge_index, batch, emb, W1, b1, W2, b2, cW1, cb1, cW2, cb2):
    x = x.astype(jnp.int32)
    edge_index = edge_index.astype(jnp.int32)
    batch = batch.astype(jnp.int32)

    xp = jnp.pad(x[:, 0], (0, NXROW * 128 - N)).reshape(NXROW, 128)
    pad_e = EP - E
    srcp = jnp.concatenate(
        [edge_index[0],
         jnp.zeros((EP + 16384 - E,), jnp.int32)]).reshape(ERX // 2, 256)
    trash = N + (jnp.arange(EP - E, dtype=jnp.int32) % (NROW - N - 128))
    dstp = jnp.concatenate(
        [edge_index[1], trash,
         jnp.zeros((16384,), jnp.int32)]).reshape(ERX // 2, 256)
    batchp = jnp.pad(batch, (0, NROW - N),
                     constant_values=2 * G).reshape(NROW // RB, 1, RB)

    h0, hist = _call_A(xp, dstp, emb)
    dinv2d, hs1 = _call_B(hist.reshape(NC * NS, NROW), h0[:NROW], W1)
    acc1 = _call_mp(hs1.reshape(NCH * NROW, CW), srcp, dstp)
    hs2 = _call_D(acc1.reshape(NCH, NROW, CW), hs1, dinv2d,
                  b1.reshape(1, H), W2)
    acc2 = _call_mp(hs2.reshape(NCH * NROW, CW), srcp, dstp)
    return _call_F(acc2.reshape(NCH, NROW, CW), hs2, dinv2d,
                   b2.reshape(1, H), batchp, cW1, cb1.reshape(1, H // 2),
                   cW2, cb2.reshape(1, 1))


# speedup vs baseline: 9.0958x; 1.0001x over previous
import jax
import jax.numpy as jnp
from jax import lax
from jax.experimental import pallas as pl
from jax.experimental.pallas import tpu as pltpu
from jax.experimental.pallas import tpu_sc as plsc

N = 50000
E = 800000
G = 128
T = 1000
D = 64
H = 128

NC = 2
NS = 16
L = 16

NROW = 53248
RB = 3328
NXROW = 512
EP = 819200
ER = EP // 128
ERT = 100
ERS = ER // NS
NCH = 4
CW = H // NCH

_mesh = plsc.VectorSubcoreMesh(core_axis_name="c", subcore_axis_name="s",
                               num_cores=NC, num_subcores=NS)


def _zero_vmem1d(ref, nelem):
    zero = jnp.zeros((L,), jnp.float32)

    def body(i, _):
        ref[pl.ds(i * L, L)] = zero
        return 0

    lax.fori_loop(0, nelem // L, body, 0)


def _zero_vmem2d(ref, nrows, ncols):
    zero = jnp.zeros((L,), jnp.float32)

    def body(i, _):
        for k in range(ncols // L):
            ref[i, pl.ds(k * L, L)] = zero
        return 0

    lax.fori_loop(0, nrows, body, 0)


def _sc_deg_embed(xp_hbm, dstp_hbm, emb_hbm, h0_hbm, hist_hbm,
                  idx_v, erows_v, hist_v, rows_v, sem):
    c = lax.axis_index("c")
    s = lax.axis_index("s")
    w = c * NS + s

    pltpu.sync_copy(dstp_hbm.at[pl.ds(w * ERT, ERT)], erows_v)
    _zero_vmem1d(hist_v, NROW)
    ones = jnp.full((L,), 1.0, jnp.float32)

    def erow(j, _):
        for k in range(256 // L):
            idx16 = erows_v[j, pl.ds(k * L, L)]
            plsc.addupdate_scatter(hist_v, [idx16], ones)
        return 0

    lax.fori_loop(0, ERT, erow, 0)
    pltpu.sync_copy(hist_v, hist_hbm.at[pl.ds(w * NROW, NROW)])

    nb = NXROW // (NC * NS)
    pltpu.sync_copy(xp_hbm.at[pl.ds(w * nb, nb)], idx_v)

    def nrow(j, _):
        pltpu.async_copy(emb_hbm.at[idx_v.at[j]], rows_v, sem).wait()
        pltpu.sync_copy(rows_v, h0_hbm.at[pl.ds((w * nb + j) * 128, 128)])
        return 0

    lax.fori_loop(0, nb, nrow, 0)


def _call_A(xp, dstp, emb):
    return pl.kernel(
        _sc_deg_embed,
        out_type=(
            jax.ShapeDtypeStruct((NXROW * 128, D), jnp.float32),
            jax.ShapeDtypeStruct((NC * NS * NROW,), jnp.float32),
        ),
        mesh=_mesh,
        scratch_types=[
            pltpu.VMEM((NXROW // (NC * NS), 128), jnp.int32),
            pltpu.VMEM((ERT, 256), jnp.int32),
            pltpu.VMEM((NROW,), jnp.float32),
            pltpu.VMEM((128, D), jnp.float32),
            pltpu.SemaphoreType.DMA,
        ],
        compiler_params=pltpu.CompilerParams(needs_layout_passes=False, use_tc_tiling_on_sc=False),
    )(xp, dstp, emb)


IDW = 256
IB = 10
ERS2 = 200
NBLK = ERS2 // IB
ERX = ER + 128


def _sc_mp(hs_hbm, srcp_hbm, dstp_hbm, acc_hbm,
           sb, db, buf0, buf1, zb_v, isem, gsem0, gsem1, zsem, acc_sp):
    c = lax.axis_index("c")
    s = lax.axis_index("s")
    _zero_vmem2d(zb_v, 32, CW)
    bufs = (buf0, buf1)
    gsems = (gsem0, gsem1)

    for k in range(2):
        ch = 2 * c + k
        off = ch * NROW

        def load_offset_ids(bb):
            base = s * ERS2 + bb * IB
            pltpu.async_copy(srcp_hbm.at[pl.ds(base, IB)], sb, isem)
            pltpu.async_copy(dstp_hbm.at[pl.ds(base, IB)], db, isem)
            pltpu.make_async_copy(srcp_hbm.at[pl.ds(base, IB)], sb,
                                  isem).wait()
            pltpu.make_async_copy(dstp_hbm.at[pl.ds(base, IB)], db,
                                  isem).wait()
            offv = jnp.full((L,), 1, jnp.int32) * off
            for r8 in range(IB):
                for k2 in range(IDW // L):
                    sb[r8, pl.ds(k2 * L, L)] = (
                        sb[r8, pl.ds(k2 * L, L)] + offv)

        def gather(r2, vb):
            pltpu.async_copy(hs_hbm.at[sb.at[r2]], bufs[vb], gsems[vb])

        def gather_wait(r2, vb):
            pltpu.make_async_copy(hs_hbm.at[sb.at[r2]], bufs[vb],
                                  gsems[vb]).wait()

        def scat(r2, vb):
            pltpu.sync_copy(bufs[vb], acc_sp.at[db.at[r2]], add=True)

        def zrow(i, _):
            pltpu.async_copy(zb_v, acc_sp.at[pl.ds(s * RB + i * 32, 32)],
                             zsem)
            return 0
        lax.fori_loop(0, RB // 32, zrow, 0)

        def zdrain(i, _):
            pltpu.make_async_copy(zb_v, acc_sp.at[pl.ds(s * RB + i * 32, 32)],
                                  zsem).wait()
            return 0
        lax.fori_loop(0, RB // 32, zdrain, 0)
        plsc.subcore_barrier()

        def block(bb, _):
            load_offset_ids(bb)
            gather(0, 0)
            for r2 in range(IB):
                gather_wait(r2, r2 % 2)
                if r2 + 1 < IB:
                    gather(r2 + 1, (r2 + 1) % 2)
                scat(r2, r2 % 2)
            return 0

        lax.fori_loop(0, NBLK, block, 0)
        plsc.subcore_barrier()

        pltpu.sync_copy(acc_sp.at[pl.ds(s * RB, RB)],
                        acc_hbm.at[pl.ds(ch * NROW + s * RB, RB)])


def _call_mp(hs_flat, srcp, dstp):
    return pl.kernel(
        _sc_mp,
        out_type=jax.ShapeDtypeStruct((NCH * NROW, CW), jnp.float32),
        mesh=_mesh,
        scratch_types=[
            pltpu.VMEM((IB, IDW), jnp.int32),
            pltpu.VMEM((IB, IDW), jnp.int32),
            pltpu.VMEM((IDW, CW), jnp.float32),
            pltpu.VMEM((IDW, CW), jnp.float32),
            pltpu.VMEM((32, CW), jnp.float32),
            pltpu.SemaphoreType.DMA,
            pltpu.SemaphoreType.DMA,
            pltpu.SemaphoreType.DMA,
            pltpu.SemaphoreType.DMA,
            pltpu.VMEM_SHARED((NROW, CW), jnp.float32),
        ],
        compiler_params=pltpu.CompilerParams(needs_layout_passes=False, use_tc_tiling_on_sc=False),
    )(hs_flat, srcp, dstp)


def _tc_prep(hist_ref, h0_ref, W1_ref, dinv_ref, hs1_ref):
    deg = jnp.sum(hist_ref[...], axis=0) + 1.0
    dinv = lax.rsqrt(deg)
    dinv_ref[0, 0, :] = dinv
    hs = jnp.dot(h0_ref[...], W1_ref[...],
                 preferred_element_type=jnp.float32) * dinv[:, None]
    for cch in range(NCH):
        hs1_ref[cch] = hs[:, cch * CW:(cch + 1) * CW]


def _call_B(hist, h0, W1):
    nblk = NROW // RB
    return pl.pallas_call(
        _tc_prep,
        grid=(nblk,),
        in_specs=[
            pl.BlockSpec((NC * NS, RB), lambda i: (0, i)),
            pl.BlockSpec((RB, D), lambda i: (i, 0)),
            pl.BlockSpec((D, H), lambda i: (0, 0)),
        ],
        out_specs=[
            pl.BlockSpec((1, 1, RB), lambda i: (i, 0, 0)),
            pl.BlockSpec((NCH, RB, CW), lambda i: (0, i, 0)),
        ],
        out_shape=[
            jax.ShapeDtypeStruct((NROW // RB, 1, RB), jnp.float32),
            jax.ShapeDtypeStruct((NCH, NROW, CW), jnp.float32),
        ],
    )(hist, h0, W1)


def _tc_mid(acc_ref, hs_ref, dinv_ref, b1_ref, W2_ref, hs2_ref):
    full = jnp.concatenate([acc_ref[cch] + hs_ref[cch] for cch in range(NCH)],
                           axis=1)
    dinv = dinv_ref[0, 0, :]
    h1 = jnp.maximum(full * dinv[:, None] + b1_ref[0, :][None, :], 0.0)
    hs2 = jnp.dot(h1, W2_ref[...],
                  preferred_element_type=jnp.float32) * dinv[:, None]
    for cch in range(NCH):
        hs2_ref[cch] = hs2[:, cch * CW:(cch + 1) * CW]


def _call_D(acc1, hs1, dinv2d, b1, W2):
    nblk = NROW // RB
    return pl.pallas_call(
        _tc_mid,
        grid=(nblk,),
        in_specs=[
            pl.BlockSpec((NCH, RB, CW), lambda i: (0, i, 0)),
            pl.BlockSpec((NCH, RB, CW), lambda i: (0, i, 0)),
            pl.BlockSpec((1, 1, RB), lambda i: (i, 0, 0)),
            pl.BlockSpec((1, H), lambda i: (0, 0)),
            pl.BlockSpec((H, H), lambda i: (0, 0)),
        ],
        out_specs=pl.BlockSpec((NCH, RB, CW), lambda i: (0, i, 0)),
        out_shape=jax.ShapeDtypeStruct((NCH, NROW, CW), jnp.float32),
    )(acc1, hs1, dinv2d, b1, W2)


def _tc_final(acc_ref, hs_ref, dinv_ref, b2_ref, batch_ref, cW1_ref, cb1_ref,
              cW2_ref, cb2_ref, out_ref, accg_ref, cnt_ref):
    i = pl.program_id(0)
    nsteps = pl.num_programs(0)

    @pl.when(i == 0)
    def _init():
        accg_ref[...] = jnp.zeros_like(accg_ref)
        cnt_ref[...] = jnp.zeros_like(cnt_ref)

    full = jnp.concatenate([acc_ref[cch] + hs_ref[cch] for cch in range(NCH)],
                           axis=1)
    dinv = dinv_ref[0, 0, :]
    out2 = full * dinv[:, None] + b2_ref[0, :][None, :]
    b = batch_ref[0, 0, :]
    seg = lax.broadcasted_iota(jnp.int32, (G, 1), 0)
    onehot = jnp.where(b[None, :] == seg, 1.0, 0.0)
    accg_ref[...] += jnp.dot(onehot, out2, preferred_element_type=jnp.float32)
    cnt_ref[...] += jnp.sum(onehot, axis=1, keepdims=True)

    @pl.when(i == nsteps - 1)
    def _fin():
        g = accg_ref[...] / jnp.maximum(cnt_ref[...], 1.0)
        z = jnp.maximum(
            jnp.dot(g, cW1_ref[...], preferred_element_type=jnp.float32)
            + cb1_ref[0, :][None, :], 0.0)
        out_ref[...] = (jnp.dot(z, cW2_ref[...],
                                preferred_element_type=jnp.float32)
                        + cb2_ref[0, :][None, :])


def _call_F(acc2, hs2, dinv2d, b2, batchp, cW1, cb1, cW2, cb2):
    nblk = NROW // RB
    return pl.pallas_call(
        _tc_final,
        grid=(nblk,),
        in_specs=[
            pl.BlockSpec((NCH, RB, CW), lambda i: (0, i, 0)),
            pl.BlockSpec((NCH, RB, CW), lambda i: (0, i, 0)),
            pl.BlockSpec((1, 1, RB), lambda i: (i, 0, 0)),
            pl.BlockSpec((1, H), lambda i: (0, 0)),
            pl.BlockSpec((1, 1, RB), lambda i: (i, 0, 0)),
            pl.BlockSpec((H, H // 2), lambda i: (0, 0)),
            pl.BlockSpec((1, H // 2), lambda i: (0, 0)),
            pl.BlockSpec((H // 2, 1), lambda i: (0, 0)),
            pl.BlockSpec((1, 1), lambda i: (0, 0)),
        ],
        out_specs=pl.BlockSpec((G, 1), lambda i: (0, 0)),
        out_shape=jax.ShapeDtypeStruct((G, 1), jnp.float32),
        scratch_shapes=[
            pltpu.VMEM((G, H), jnp.float32),
            pltpu.VMEM((G, 1), jnp.float32),
        ],
    )(acc2, hs2, dinv2d, b2, batchp, cW1, cb1, cW2, cb2)


def kernel(x, edge_index, batch, emb, W1, b1, W2, b2, cW1, cb1, cW2, cb2):
    x = x.astype(jnp.int32)
    edge_index = edge_index.astype(jnp.int32)
    batch = batch.astype(jnp.int32)

    xp = jnp.pad(x[:, 0], (0, NXROW * 128 - N)).reshape(NXROW, 128)
    pad_e = EP - E
    srcp = jnp.concatenate(
        [edge_index[0],
         jnp.zeros((EP + 16384 - E,), jnp.int32)]).reshape(ERX // 2, 256)
    trash = N + (jnp.arange(EP - E, dtype=jnp.int32) % (NROW - N - 128))
    dstp = jnp.concatenate(
        [edge_index[1], trash,
         jnp.zeros((16384,), jnp.int32)]).reshape(ERX // 2, 256)
    batchp = jnp.pad(batch, (0, NROW - N),
                     constant_values=2 * G).reshape(NROW // RB, 1, RB)

    h0, hist = _call_A(xp, dstp, emb)
    dinv2d, hs1 = _call_B(hist.reshape(NC * NS, NROW), h0[:NROW], W1)
    acc1 = _call_mp(hs1.reshape(NCH * NROW, CW), srcp, dstp)
    hs2 = _call_D(acc1.reshape(NCH, NROW, CW), hs1, dinv2d,
                  b1.reshape(1, H), W2)
    acc2 = _call_mp(hs2.reshape(NCH * NROW, CW), srcp, dstp)
    return _call_F(acc2.reshape(NCH, NROW, CW), hs2, dinv2d,
                   b2.reshape(1, H), batchp, cW1, cb1.reshape(1, H // 2),
                   cW2, cb2.reshape(1, 1))
